# baseline (speedup 1.0000x reference)
import functools

import numpy as np
import jax
import jax.numpy as jnp
from jax import lax
from jax.experimental import pallas as pl
from jax.experimental.pallas import tpu as pltpu
from jax.experimental.pallas import tpu_sc as plsc

_VOCAB = 50257
_D = 1024
_CTX = 2048
_B = 4
_S = 2048

_NC = 2
_NS = 16
_NW = _NC * _NS
_PW = _S // _NW
_CP = 8
_NH = _PW // _CP
_NB = 3


def _sin_table():
    pos = np.arange(_CTX)[:, None].astype(np.float32)
    i = np.arange(_D)[None, :].astype(np.float32)
    rates = 1.0 / np.power(10000.0, (2.0 * np.floor(i / 2.0)) / np.float32(_D))
    angles = pos * rates
    pe = np.zeros((_CTX, _D), dtype=np.float32)
    pe[:, 0::2] = np.sin(angles[:, 0::2])
    pe[:, 1::2] = np.cos(angles[:, 1::2])
    return pe


_PE = _sin_table()

_mesh = plsc.VectorSubcoreMesh(core_axis_name="c", subcore_axis_name="s")


@functools.partial(
    pl.kernel,
    mesh=_mesh,
    out_type=jax.ShapeDtypeStruct((_B, _S, _D), jnp.float32),
    scratch_types=[
        pltpu.VMEM((_NH, _B * _CP), jnp.int32),
        pltpu.VMEM((_NB, _CP, _D), jnp.float32),
        pltpu.VMEM((_NB, _B * _CP, _D), jnp.float32),
        pltpu.SemaphoreType.DMA,
        pltpu.SemaphoreType.DMA,
        pltpu.SemaphoreType.DMA,
        pltpu.SemaphoreType.DMA,
        pltpu.SemaphoreType.DMA,
        pltpu.SemaphoreType.DMA,
        pltpu.SemaphoreType.DMA,
        pltpu.SemaphoreType.DMA,
        pltpu.SemaphoreType.DMA,
        pltpu.SemaphoreType.DMA,
    ],
)
def _embed(ids_hbm, table_hbm, pe_hbm, out_hbm, idx_v, pe_v, rows_v,
           sg0, sg1, sg2, sp0, sp1, sp2, ss0, ss1, ss2, si):
    wid = lax.axis_index("s") * _NC + lax.axis_index("c")
    p0 = wid * _PW
    sg = (sg0, sg1, sg2)
    sp = (sp0, sp1, sp2)
    ss = (ss0, ss1, ss2)

    idesc = []
    for h in range(_NH):
        for b in range(_B):
            idesc.append(pltpu.async_copy(
                ids_hbm.at[b, pl.ds(p0 + h * _CP, _CP)],
                idx_v.at[h, pl.ds(b * _CP, _CP)], si))

    g_desc = [None] * _NB
    p_desc = [None] * _NB
    s_desc = [None] * _NB

    def issue(h):
        nb = h % _NB
        ps = p0 + h * _CP
        p_desc[nb] = pltpu.async_copy(
            pe_hbm.at[pl.ds(ps, _CP)], pe_v.at[nb], sp[nb])
        g_desc[nb] = pltpu.async_copy(
            table_hbm.at[idx_v.at[h]], rows_v.at[nb], sg[nb])

    for d in idesc[: 2 * _B]:
        d.wait()
    issue(0)
    issue(1)
    for d in idesc[2 * _B:]:
        d.wait()
    for h in range(_NH):
        nb = h % _NB
        p_desc[nb].wait()
        g_desc[nb].wait()

        @plsc.parallel_loop(0, _CP * (_D // 64), unroll=2)
        def _add(i):
            r = i >> 4
            c = i & 15
            for cc in range(4):
                sl = pl.ds((c * 4 + cc) * 16, 16)
                pvec = pe_v[nb, r, sl]
                for b in range(_B):
                    plsc.addupdate(rows_v.at[nb, b * _CP + r, sl], pvec)

        ps = p0 + h * _CP
        s_desc[nb] = [
            pltpu.async_copy(
                rows_v.at[nb, pl.ds(b * _CP, _CP)],
                out_hbm.at[b, pl.ds(ps, _CP)], ss[nb])
            for b in range(_B)
        ]
        if h + 2 < _NH:
            ob = (h + 2) % _NB
            if s_desc[ob] is not None:
                for d in s_desc[ob]:
                    d.wait()
            issue(h + 2)

    for sd in s_desc:
        if sd is not None:
            for d in sd:
                d.wait()


def kernel(input_ids, token_table):
    return _embed(input_ids, token_table, _PE)


# speedup vs baseline: 1.0269x; 1.0269x over previous
import functools

import numpy as np
import jax
import jax.numpy as jnp
from jax import lax
from jax.experimental import pallas as pl
from jax.experimental.pallas import tpu as pltpu
from jax.experimental.pallas import tpu_sc as plsc

_VOCAB = 50257
_D = 1024
_CTX = 2048
_B = 4
_S = 2048

_NC = 2
_NS = 16
_NW = _NC * _NS
_PW = _S // _NW
_CP = 8
_NH = _PW // _CP
_NB = 3


def _sin_table():
    pos = np.arange(_CTX)[:, None].astype(np.float32)
    i = np.arange(_D)[None, :].astype(np.float32)
    rates = 1.0 / np.power(10000.0, (2.0 * np.floor(i / 2.0)) / np.float32(_D))
    angles = pos * rates
    pe = np.zeros((_CTX, _D), dtype=np.float32)
    pe[:, 0::2] = np.sin(angles[:, 0::2])
    pe[:, 1::2] = np.cos(angles[:, 1::2])
    return pe


_PE = _sin_table()

_mesh = plsc.VectorSubcoreMesh(core_axis_name="c", subcore_axis_name="s")


@functools.partial(
    pl.kernel,
    mesh=_mesh,
    out_type=jax.ShapeDtypeStruct((_B, _S, _D), jnp.float32),
    scratch_types=[
        pltpu.VMEM((_NH, _B * _CP), jnp.int32),
        pltpu.VMEM((_NB, _CP, _D), jnp.float32),
        pltpu.VMEM((_NB, _B * _CP, _D), jnp.float32),
        pltpu.SemaphoreType.DMA,
        pltpu.SemaphoreType.DMA,
        pltpu.SemaphoreType.DMA,
        pltpu.SemaphoreType.DMA,
        pltpu.SemaphoreType.DMA,
        pltpu.SemaphoreType.DMA,
        pltpu.SemaphoreType.DMA,
        pltpu.SemaphoreType.DMA,
        pltpu.SemaphoreType.DMA,
        pltpu.SemaphoreType.DMA,
    ],
)
def _embed(ids_hbm, table_hbm, pe_hbm, out_hbm, idx_v, pe_v, rows_v,
           sg0, sg1, sg2, sp0, sp1, sp2, ss0, ss1, ss2, si):
    wid = lax.axis_index("s") * _NC + lax.axis_index("c")
    p0 = wid * _PW
    sg = (sg0, sg1, sg2)
    sp = (sp0, sp1, sp2)
    ss = (ss0, ss1, ss2)

    idesc = []
    for h in range(_NH):
        for b in range(_B):
            idesc.append(pltpu.async_copy(
                ids_hbm.at[b, pl.ds(p0 + h * _CP, _CP)],
                idx_v.at[h, pl.ds(b * _CP, _CP)], si))

    g_desc = [None] * _NB
    p_desc = [None] * _NB
    s_desc = [None] * _NB

    def issue(h):
        nb = h % _NB
        ps = p0 + h * _CP
        p_desc[nb] = pltpu.async_copy(
            pe_hbm.at[pl.ds(ps, _CP)], pe_v.at[nb], sp[nb])
        g_desc[nb] = [
            pltpu.async_copy(
                table_hbm.at[idx_v.at[h, pl.ds(half * 2 * _CP, 2 * _CP)]],
                rows_v.at[nb, pl.ds(half * 2 * _CP, 2 * _CP)], sg[nb])
            for half in range(2)
        ]

    for d in idesc[: 2 * _B]:
        d.wait()
    issue(0)
    issue(1)
    for d in idesc[2 * _B:]:
        d.wait()
    for h in range(_NH):
        nb = h % _NB
        p_desc[nb].wait()
        ps = p0 + h * _CP
        sd = []
        for half in range(2):
            g_desc[nb][half].wait()

            @plsc.parallel_loop(0, _CP * (_D // 64), unroll=2)
            def _add(i, _half=half):
                r = i >> 4
                c = i & 15
                for cc in range(4):
                    sl = pl.ds((c * 4 + cc) * 16, 16)
                    pvec = pe_v[nb, r, sl]
                    for b in (2 * _half, 2 * _half + 1):
                        plsc.addupdate(rows_v.at[nb, b * _CP + r, sl], pvec)

            for b in (2 * half, 2 * half + 1):
                sd.append(pltpu.async_copy(
                    rows_v.at[nb, pl.ds(b * _CP, _CP)],
                    out_hbm.at[b, pl.ds(ps, _CP)], ss[nb]))
        s_desc[nb] = sd
        if h + 2 < _NH:
            ob = (h + 2) % _NB
            if s_desc[ob] is not None:
                for d in s_desc[ob]:
                    d.wait()
            issue(h + 2)

    for sd in s_desc:
        if sd is not None:
            for d in sd:
                d.wait()


def kernel(input_ids, token_table):
    return _embed(input_ids, token_table, _PE)
